# baseline (speedup 1.0000x reference)
import functools

import jax
import jax.numpy as jnp
from jax import lax
from jax.experimental import pallas as pl
from jax.experimental.pallas import tpu as pltpu, tpu_sc as plsc

NC = 2
NS = 16
LANES = 16
NW = NC * NS


def _make_kernel(n_shots, n_seasons, max_weeks):
    b_per_w = n_shots // NW
    CH = 2048
    n_sub = b_per_w // CH

    mesh = plsc.VectorSubcoreMesh(
        core_axis_name="c", subcore_axis_name="s",
        num_cores=NC, num_subcores=NS)

    @functools.partial(
        pl.kernel,
        out_type=jax.ShapeDtypeStruct((n_shots,), jnp.float32),
        mesh=mesh,
        scratch_types=[
            pltpu.VMEM((CH,), jnp.int32),
            pltpu.VMEM((CH,), jnp.int32),
            pltpu.VMEM((CH,), jnp.int32),
            pltpu.VMEM((CH,), jnp.int32),
            pltpu.VMEM((CH,), jnp.float32),
            pltpu.VMEM((CH,), jnp.float32),
            pltpu.VMEM((CH,), jnp.float32),
            pltpu.VMEM((2 * LANES,), jnp.float32),
            pltpu.SemaphoreType.DMA,
            pltpu.SemaphoreType.DMA,
        ],
    )
    def irt_kernel(theta_h, phi_h, xg_h, scal_h, sh_h, go_h, se_h, wk_h,
                   out_h,
                   sh_v, go_v, se_v, wk_v, th_v, ph_v, xg_v, scal_v,
                   sem_in, sem_g):
        wid = lax.axis_index("s") * NC + lax.axis_index("c")
        base = wid * b_per_w

        pltpu.sync_copy(scal_h, scal_v)
        b0v = scal_v[pl.ds(0, LANES)]
        alv = scal_v[pl.ds(LANES, LANES)]

        def chunk(c, carry):
            off = pl.multiple_of(base + c * CH, CH)
            cps = [
                pltpu.async_copy(sh_h.at[pl.ds(off, CH)], sh_v, sem_in),
                pltpu.async_copy(go_h.at[pl.ds(off, CH)], go_v, sem_in),
                pltpu.async_copy(se_h.at[pl.ds(off, CH)], se_v, sem_in),
                pltpu.async_copy(wk_h.at[pl.ds(off, CH)], wk_v, sem_in),
                pltpu.async_copy(xg_h.at[pl.ds(off, CH)], xg_v, sem_in),
            ]
            for cp in cps:
                cp.wait()

            @plsc.parallel_loop(0, CH, LANES, unroll=8)
            def ixbody(i):
                sl = pl.ds(pl.multiple_of(i, LANES), LANES)
                sev = se_v[sl]
                wkv = wk_v[sl]
                sh_v[sl] = (sh_v[sl] * n_seasons + sev) * max_weeks + wkv
                go_v[sl] = (go_v[sl] * n_seasons + sev) * max_weeks + wkv

            g1 = pltpu.async_copy(theta_h.at[sh_v], th_v, sem_g)
            g2 = pltpu.async_copy(phi_h.at[go_v], ph_v, sem_g)
            g1.wait()
            g2.wait()

            @plsc.parallel_loop(0, CH, LANES, unroll=8)
            def cmb(i):
                sl = pl.ds(pl.multiple_of(i, LANES), LANES)
                xg_v[sl] = b0v + alv * xg_v[sl] + th_v[sl] - ph_v[sl]

            pltpu.sync_copy(xg_v, out_h.at[pl.ds(off, CH)])
            return carry

        lax.fori_loop(0, n_sub, chunk, 0)

    return irt_kernel


def kernel(xg_logit, theta, phi, beta0, alpha, shooter_idx, goalie_idx,
           season_idx, week_idx):
    n_shooters, n_seasons, max_weeks = theta.shape
    n_shots = xg_logit.shape[0]
    scal = jnp.concatenate([
        jnp.broadcast_to(beta0, (LANES,)),
        jnp.broadcast_to(alpha, (LANES,)),
    ])
    k = _make_kernel(n_shots, n_seasons, max_weeks)
    t1 = theta.reshape(-1)
    p1 = phi.reshape(-1)
    return k(t1, p1, xg_logit, scal,
             shooter_idx, goalie_idx, season_idx, week_idx)


# speedup vs baseline: 1.0206x; 1.0206x over previous
import functools

import jax
import jax.numpy as jnp
from jax import lax
from jax.experimental import pallas as pl
from jax.experimental.pallas import tpu as pltpu, tpu_sc as plsc

NC = 2
NS = 16
LANES = 16
NW = NC * NS


def _make_kernel(n_shots, n_seasons, max_weeks):
    b_per_w = n_shots // NW
    CH = 8192
    n_sub = b_per_w // CH

    mesh = plsc.VectorSubcoreMesh(
        core_axis_name="c", subcore_axis_name="s",
        num_cores=NC, num_subcores=NS)

    @functools.partial(
        pl.kernel,
        out_type=jax.ShapeDtypeStruct((n_shots,), jnp.float32),
        mesh=mesh,
        scratch_types=[
            pltpu.VMEM((CH,), jnp.int32),
            pltpu.VMEM((CH,), jnp.int32),
            pltpu.VMEM((CH,), jnp.int32),
            pltpu.VMEM((CH,), jnp.int32),
            pltpu.VMEM((CH,), jnp.float32),
            pltpu.VMEM((CH,), jnp.float32),
            pltpu.VMEM((CH,), jnp.float32),
            pltpu.VMEM((2 * LANES,), jnp.float32),
            pltpu.SemaphoreType.DMA,
            pltpu.SemaphoreType.DMA,
        ],
    )
    def irt_kernel(theta_h, phi_h, xg_h, scal_h, sh_h, go_h, se_h, wk_h,
                   out_h,
                   sh_v, go_v, se_v, wk_v, th_v, ph_v, xg_v, scal_v,
                   sem_in, sem_g):
        wid = lax.axis_index("s") * NC + lax.axis_index("c")
        base = wid * b_per_w

        pltpu.sync_copy(scal_h, scal_v)
        b0v = scal_v[pl.ds(0, LANES)]
        alv = scal_v[pl.ds(LANES, LANES)]

        def chunk(c, carry):
            off = pl.multiple_of(base + c * CH, CH)
            cps = [
                pltpu.async_copy(sh_h.at[pl.ds(off, CH)], sh_v, sem_in),
                pltpu.async_copy(go_h.at[pl.ds(off, CH)], go_v, sem_in),
                pltpu.async_copy(se_h.at[pl.ds(off, CH)], se_v, sem_in),
                pltpu.async_copy(wk_h.at[pl.ds(off, CH)], wk_v, sem_in),
                pltpu.async_copy(xg_h.at[pl.ds(off, CH)], xg_v, sem_in),
            ]
            for cp in cps:
                cp.wait()

            @plsc.parallel_loop(0, CH, LANES, unroll=8)
            def ixbody(i):
                sl = pl.ds(pl.multiple_of(i, LANES), LANES)
                sev = se_v[sl]
                wkv = wk_v[sl]
                sh_v[sl] = (sh_v[sl] * n_seasons + sev) * max_weeks + wkv
                go_v[sl] = (go_v[sl] * n_seasons + sev) * max_weeks + wkv

            g1 = pltpu.async_copy(theta_h.at[sh_v], th_v, sem_g)
            g2 = pltpu.async_copy(phi_h.at[go_v], ph_v, sem_g)
            g1.wait()
            g2.wait()

            @plsc.parallel_loop(0, CH, LANES, unroll=8)
            def cmb(i):
                sl = pl.ds(pl.multiple_of(i, LANES), LANES)
                xg_v[sl] = b0v + alv * xg_v[sl] + th_v[sl] - ph_v[sl]

            pltpu.sync_copy(xg_v, out_h.at[pl.ds(off, CH)])
            return carry

        lax.fori_loop(0, n_sub, chunk, 0)

    return irt_kernel


def kernel(xg_logit, theta, phi, beta0, alpha, shooter_idx, goalie_idx,
           season_idx, week_idx):
    n_shooters, n_seasons, max_weeks = theta.shape
    n_shots = xg_logit.shape[0]
    scal = jnp.concatenate([
        jnp.broadcast_to(beta0, (LANES,)),
        jnp.broadcast_to(alpha, (LANES,)),
    ])
    k = _make_kernel(n_shots, n_seasons, max_weeks)
    t1 = theta.reshape(-1)
    p1 = phi.reshape(-1)
    return k(t1, p1, xg_logit, scal,
             shooter_idx, goalie_idx, season_idx, week_idx)


# speedup vs baseline: 1.0289x; 1.0081x over previous
import functools

import jax
import jax.numpy as jnp
from jax import lax
from jax.experimental import pallas as pl
from jax.experimental.pallas import tpu as pltpu, tpu_sc as plsc

NC = 2
NS = 16
LANES = 16
NW = NC * NS


def _make_kernel(n_shots, n_seasons, max_weeks):
    b_per_w = n_shots // NW
    CH = 2048
    n_sub = b_per_w // CH

    mesh = plsc.VectorSubcoreMesh(
        core_axis_name="c", subcore_axis_name="s",
        num_cores=NC, num_subcores=NS)

    ibuf = [pltpu.VMEM((CH,), jnp.int32)] * 8
    fbuf = [pltpu.VMEM((CH,), jnp.float32)] * 6

    @functools.partial(
        pl.kernel,
        out_type=jax.ShapeDtypeStruct((n_shots,), jnp.float32),
        mesh=mesh,
        scratch_types=ibuf + fbuf + [
            pltpu.VMEM((2 * LANES,), jnp.float32),
            pltpu.SemaphoreType.DMA,
            pltpu.SemaphoreType.DMA,
            pltpu.SemaphoreType.DMA,
            pltpu.SemaphoreType.DMA,
        ],
    )
    def irt_kernel(theta_h, phi_h, xg_h, scal_h, sh_h, go_h, se_h, wk_h,
                   out_h,
                   sh0, go0, se0, wk0, sh1, go1, se1, wk1,
                   th0, ph0, xg0, th1, ph1, xg1,
                   scal_v, sem_i0, sem_i1, sem_g0, sem_g1):
        wid = lax.axis_index("s") * NC + lax.axis_index("c")
        base = wid * b_per_w

        pltpu.sync_copy(scal_h, scal_v)
        b0v = scal_v[pl.ds(0, LANES)]
        alv = scal_v[pl.ds(LANES, LANES)]

        bufs = [
            (sh0, go0, se0, wk0, th0, ph0, xg0, sem_i0, sem_g0),
            (sh1, go1, se1, wk1, th1, ph1, xg1, sem_i1, sem_g1),
        ]

        def start_in(c):
            sh_v, go_v, se_v, wk_v, _, _, xg_v, sem_in, _ = bufs[c % 2]
            off = pl.multiple_of(base + c * CH, CH)
            return [
                pltpu.async_copy(sh_h.at[pl.ds(off, CH)], sh_v, sem_in),
                pltpu.async_copy(go_h.at[pl.ds(off, CH)], go_v, sem_in),
                pltpu.async_copy(se_h.at[pl.ds(off, CH)], se_v, sem_in),
                pltpu.async_copy(wk_h.at[pl.ds(off, CH)], wk_v, sem_in),
                pltpu.async_copy(xg_h.at[pl.ds(off, CH)], xg_v, sem_in),
            ]

        def idx_and_gather(c):
            sh_v, go_v, se_v, wk_v, th_v, ph_v, _, _, sem_g = bufs[c % 2]

            @plsc.parallel_loop(0, CH, LANES, unroll=8)
            def ixbody(i):
                sl = pl.ds(pl.multiple_of(i, LANES), LANES)
                sev = se_v[sl]
                wkv = wk_v[sl]
                sh_v[sl] = (sh_v[sl] * n_seasons + sev) * max_weeks + wkv
                go_v[sl] = (go_v[sl] * n_seasons + sev) * max_weeks + wkv

            return [
                pltpu.async_copy(theta_h.at[sh_v], th_v, sem_g),
                pltpu.async_copy(phi_h.at[go_v], ph_v, sem_g),
            ]

        def combine_and_out(c):
            _, _, _, _, th_v, ph_v, xg_v, _, _ = bufs[c % 2]

            @plsc.parallel_loop(0, CH, LANES, unroll=8)
            def cmb(i):
                sl = pl.ds(pl.multiple_of(i, LANES), LANES)
                xg_v[sl] = b0v + alv * xg_v[sl] + th_v[sl] - ph_v[sl]

            off = pl.multiple_of(base + c * CH, CH)
            pltpu.sync_copy(xg_v, out_h.at[pl.ds(off, CH)])

        for cp in start_in(0):
            cp.wait()
        gathers = {0: idx_and_gather(0)}
        pend_in = {1: start_in(1)} if n_sub > 1 else {}
        for c in range(n_sub):
            if c + 1 < n_sub:
                for cp in pend_in.pop(c + 1):
                    cp.wait()
                gathers[c + 1] = idx_and_gather(c + 1)
            for cp in gathers.pop(c):
                cp.wait()
            combine_and_out(c)
            if c + 2 < n_sub:
                pend_in[c + 2] = start_in(c + 2)

    return irt_kernel


def kernel(xg_logit, theta, phi, beta0, alpha, shooter_idx, goalie_idx,
           season_idx, week_idx):
    n_shooters, n_seasons, max_weeks = theta.shape
    n_shots = xg_logit.shape[0]
    scal = jnp.concatenate([
        jnp.broadcast_to(beta0, (LANES,)),
        jnp.broadcast_to(alpha, (LANES,)),
    ])
    k = _make_kernel(n_shots, n_seasons, max_weeks)
    t1 = theta.reshape(-1)
    p1 = phi.reshape(-1)
    return k(t1, p1, xg_logit, scal,
             shooter_idx, goalie_idx, season_idx, week_idx)


# speedup vs baseline: 1.0294x; 1.0005x over previous
import functools

import jax
import jax.numpy as jnp
from jax import lax
from jax.experimental import pallas as pl
from jax.experimental.pallas import tpu as pltpu, tpu_sc as plsc

NC = 2
NS = 16
LANES = 16
NW = NC * NS
DEPTH = 4


def _make_kernel(n_shots, n_seasons, max_weeks):
    b_per_w = n_shots // NW
    CH = 2048
    n_sub = b_per_w // CH

    mesh = plsc.VectorSubcoreMesh(
        core_axis_name="c", subcore_axis_name="s",
        num_cores=NC, num_subcores=NS)

    ring = ([pltpu.VMEM((CH,), jnp.int32)] * 4
            + [pltpu.VMEM((CH,), jnp.float32)] * 3
            + [pltpu.SemaphoreType.DMA] * 2) * DEPTH

    @functools.partial(
        pl.kernel,
        out_type=jax.ShapeDtypeStruct((n_shots,), jnp.float32),
        mesh=mesh,
        scratch_types=ring + [pltpu.VMEM((2 * LANES,), jnp.float32)],
    )
    def irt_kernel(theta_h, phi_h, xg_h, scal_h, sh_h, go_h, se_h, wk_h,
                   out_h, *scratch):
        bufs = [tuple(scratch[9 * d:9 * (d + 1)]) for d in range(DEPTH)]
        scal_v = scratch[9 * DEPTH]

        wid = lax.axis_index("s") * NC + lax.axis_index("c")
        base = wid * b_per_w

        pltpu.sync_copy(scal_h, scal_v)
        b0v = scal_v[pl.ds(0, LANES)]
        alv = scal_v[pl.ds(LANES, LANES)]

        def start_in(c):
            sh_v, go_v, se_v, wk_v, _, _, xg_v, sem_in, _ = bufs[c % DEPTH]
            off = pl.multiple_of(base + c * CH, CH)
            return [
                pltpu.async_copy(sh_h.at[pl.ds(off, CH)], sh_v, sem_in),
                pltpu.async_copy(go_h.at[pl.ds(off, CH)], go_v, sem_in),
                pltpu.async_copy(se_h.at[pl.ds(off, CH)], se_v, sem_in),
                pltpu.async_copy(wk_h.at[pl.ds(off, CH)], wk_v, sem_in),
                pltpu.async_copy(xg_h.at[pl.ds(off, CH)], xg_v, sem_in),
            ]

        def idx_and_gather(c):
            sh_v, go_v, se_v, wk_v, th_v, ph_v, _, _, sem_g = bufs[c % DEPTH]

            @plsc.parallel_loop(0, CH, LANES, unroll=8)
            def ixbody(i):
                sl = pl.ds(pl.multiple_of(i, LANES), LANES)
                sev = se_v[sl]
                wkv = wk_v[sl]
                sh_v[sl] = (sh_v[sl] * n_seasons + sev) * max_weeks + wkv
                go_v[sl] = (go_v[sl] * n_seasons + sev) * max_weeks + wkv

            return [
                pltpu.async_copy(theta_h.at[sh_v], th_v, sem_g),
                pltpu.async_copy(phi_h.at[go_v], ph_v, sem_g),
            ]

        def combine_and_out(c):
            _, _, _, _, th_v, ph_v, xg_v, _, _ = bufs[c % DEPTH]

            @plsc.parallel_loop(0, CH, LANES, unroll=8)
            def cmb(i):
                sl = pl.ds(pl.multiple_of(i, LANES), LANES)
                xg_v[sl] = b0v + alv * xg_v[sl] + th_v[sl] - ph_v[sl]

            off = pl.multiple_of(base + c * CH, CH)
            pltpu.sync_copy(xg_v, out_h.at[pl.ds(off, CH)])

        pend_in = {}
        for c in range(min(DEPTH - 1, n_sub)):
            pend_in[c] = start_in(c)
        gathers = {}
        for c in range(min(2, n_sub)):
            for cp in pend_in.pop(c):
                cp.wait()
            gathers[c] = idx_and_gather(c)
        for c in range(n_sub):
            if c + 2 < n_sub:
                for cp in pend_in.pop(c + 2):
                    cp.wait()
                gathers[c + 2] = idx_and_gather(c + 2)
            for cp in gathers.pop(c):
                cp.wait()
            combine_and_out(c)
            if c + DEPTH - 1 < n_sub:
                pend_in[c + DEPTH - 1] = start_in(c + DEPTH - 1)

    return irt_kernel


def kernel(xg_logit, theta, phi, beta0, alpha, shooter_idx, goalie_idx,
           season_idx, week_idx):
    n_shooters, n_seasons, max_weeks = theta.shape
    n_shots = xg_logit.shape[0]
    scal = jnp.concatenate([
        jnp.broadcast_to(beta0, (LANES,)),
        jnp.broadcast_to(alpha, (LANES,)),
    ])
    k = _make_kernel(n_shots, n_seasons, max_weeks)
    t1 = theta.reshape(-1)
    p1 = phi.reshape(-1)
    return k(t1, p1, xg_logit, scal,
             shooter_idx, goalie_idx, season_idx, week_idx)
